# baseline (speedup 1.0000x reference)
import jax
import jax.numpy as jnp
from jax.experimental import pallas as pl
from jax.experimental.pallas import tpu as pltpu

_NE = 64
_B = 32768
_K = 2
_ROWS = 8192
_GRID = _B // _ROWS


def _body(probs_ref, idx_ref, out_ref, acc_ref):
    i = pl.program_id(0)

    @pl.when(i == 0)
    def _init():
        acc_ref[...] = jnp.zeros_like(acc_ref)

    acc_ref[...] += jnp.sum(probs_ref[...], axis=0, keepdims=True)
    acc_ref[...] += jnp.sum(idx_ref[...]).astype(jnp.float32)

    @pl.when(i == _GRID - 1)
    def _fini():
        out_ref[0, 0] = jnp.sum(acc_ref[...])


def kernel(router_probs, expert_indices):
    idx = expert_indices.astype(jnp.int32)
    out = pl.pallas_call(
        _body,
        grid=(_GRID,),
        in_specs=[
            pl.BlockSpec((_ROWS, _NE), lambda i: (i, 0)),
            pl.BlockSpec((_ROWS, _K), lambda i: (i, 0)),
        ],
        out_specs=pl.BlockSpec((1, 1), lambda i: (0, 0),
                               memory_space=pltpu.SMEM),
        out_shape=jax.ShapeDtypeStruct((1, 1), jnp.float32),
        scratch_shapes=[pltpu.VMEM((1, _NE), jnp.float32)],
    )(router_probs, idx)
    return out[0, 0]


# speedup vs baseline: 6.2185x; 6.2185x over previous
import jax
import jax.numpy as jnp
from jax.experimental import pallas as pl
from jax.experimental.pallas import tpu as pltpu

_NE = 64
_ALPHA = 0.01
_B = 32768
_K = 2
_COLS = 8192
_GRID = _B // _COLS


def _body(probs_ref, idx_ref, out_ref, acc_ref, cnt_ref):
    i = pl.program_id(0)

    @pl.when(i == 0)
    def _init():
        acc_ref[...] = jnp.zeros_like(acc_ref)
        cnt_ref[...] = jnp.zeros_like(cnt_ref)

    a = acc_ref[...]
    for c in range(_COLS // 128):
        a = a + probs_ref[:, 128 * c:128 * (c + 1)]
    acc_ref[...] = a

    ciota = jax.lax.broadcasted_iota(jnp.int32, (8, 1), 0)
    dims = (((1,), (1,)), ((), ()))
    for r in range(_K):
        row = idx_ref[r:r + 1, :]
        coarse = ((row >> 3) == ciota).astype(jnp.float32)
        fine = ((row & 7) == ciota).astype(jnp.float32)
        cnt_ref[...] += jax.lax.dot_general(
            fine, coarse, dims, preferred_element_type=jnp.float32)

    @pl.when(i == _GRID - 1)
    def _fini():
        colsum = jnp.sum(acc_ref[...], axis=1, keepdims=True)
        cs_t = jnp.concatenate(
            [colsum[8 * c:8 * (c + 1), :] for c in range(8)], axis=1)
        scale = (_ALPHA * _NE) / (_B * float(_B * _K))
        out_ref[0, 0] = scale * jnp.sum(cs_t * cnt_ref[...])


def kernel(router_probs, expert_indices):
    probs_t = jnp.swapaxes(router_probs, 0, 1)
    idx_t = jnp.swapaxes(expert_indices, 0, 1).astype(jnp.int32)
    out = pl.pallas_call(
        _body,
        grid=(_GRID,),
        in_specs=[
            pl.BlockSpec((_NE, _COLS), lambda i: (0, i)),
            pl.BlockSpec((_K, _COLS), lambda i: (0, i)),
        ],
        out_specs=pl.BlockSpec((1, 1), lambda i: (0, 0),
                               memory_space=pltpu.SMEM),
        out_shape=jax.ShapeDtypeStruct((1, 1), jnp.float32),
        scratch_shapes=[
            pltpu.VMEM((_NE, 128), jnp.float32),
            pltpu.VMEM((8, 8), jnp.float32),
        ],
    )(probs_t, idx_t)
    return out[0, 0]
